# Initial kernel scaffold; baseline (speedup 1.0000x reference)
import jax
import jax.numpy as jnp
from jax.experimental import pallas as pl


def kernel(x, edge_index, batch, W1, b1, W2, b2, Wfc, bfc):
    raise NotImplementedError("write your pallas kernel here")


# speedup vs baseline: 16.2198x; 16.2198x over previous
import functools

import jax
import jax.numpy as jnp
from jax import lax
from jax.experimental import pallas as pl
from jax.experimental.pallas import tpu as pltpu
from jax.experimental.pallas import tpu_sc as plsc

N = 10000
E = 320000
D = 128
G = 64

NC = 2
NS = 16
NW = NC * NS
CHUNK = 128
NCH = 80
NCH_ALL = NCH + 2
EPT = NCH * CHUNK
EPAD = NW * EPT
NPAD = 10240
RPT = NPAD // NS
NB = NPAD // 128
F32 = jnp.float32


def _deg_body(idx_hbm, zeros_hbm, out_hbm, ring, deg_v, isem_a, isem_b):
    c = lax.axis_index("c")
    s = lax.axis_index("s")
    wid = s * NC + c
    pltpu.sync_copy(zeros_hbm, deg_v)
    ones = jnp.full((16,), 1.0, F32)

    pltpu.async_copy(idx_hbm.at[wid, 0], ring.at[0], isem_a)
    pltpu.async_copy(idx_hbm.at[wid, 1], ring.at[1], isem_b)

    def pair(i, carry):
        j0 = 2 * i
        for b, isem in ((0, isem_a), (1, isem_b)):
            j = j0 + b
            pltpu.make_async_copy(idx_hbm.at[wid, 0], ring.at[b], isem).wait()
            for k in range(CHUNK // 16):
                idxv = ring[b, 1, pl.ds(k * 16, 16)]
                plsc.addupdate_scatter(deg_v, [idxv], ones)
            pltpu.async_copy(idx_hbm.at[wid, j + 2], ring.at[b], isem)
        return carry

    lax.fori_loop(0, NCH // 2, pair, 0)
    pltpu.make_async_copy(idx_hbm.at[wid, 0], ring.at[0], isem_a).wait()
    pltpu.make_async_copy(idx_hbm.at[wid, 0], ring.at[1], isem_b).wait()
    pltpu.sync_copy(deg_v, out_hbm.at[wid])


@functools.cache
def _deg_kernel():
    return pl.kernel(
        _deg_body,
        out_type=jax.ShapeDtypeStruct((NW, NPAD), F32),
        mesh=plsc.VectorSubcoreMesh(core_axis_name="c", subcore_axis_name="s",
                                    num_cores=NC, num_subcores=NS),
        scratch_types=[
            pltpu.VMEM((2, 2, CHUNK), jnp.int32),
            pltpu.VMEM((NPAD,), F32),
            pltpu.SemaphoreType.DMA,
            pltpu.SemaphoreType.DMA,
        ],
        compiler_params=pltpu.CompilerParams(needs_layout_passes=False),
    )


def _agg_body(p_hbm, idx_hbm, zeros_hbm, out_hbm,
              ring, buf_a, buf_b, isem_a, isem_b, sem_a, sem_b, agg_s):
    c = lax.axis_index("c")
    s = lax.axis_index("s")
    wid = s * NC + c
    rows = pl.ds(s * RPT, RPT)
    pltpu.sync_copy(zeros_hbm.at[rows], agg_s.at[rows])
    plsc.subcore_barrier()

    pltpu.async_copy(idx_hbm.at[wid, 0], ring.at[0], isem_a)
    pltpu.async_copy(idx_hbm.at[wid, 1], ring.at[1], isem_b)
    pltpu.make_async_copy(idx_hbm.at[wid, 0], ring.at[0], isem_a).wait()
    pltpu.async_copy(p_hbm.at[ring.at[0, 0]], buf_a, sem_a)
    pltpu.make_async_copy(idx_hbm.at[wid, 1], ring.at[1], isem_b).wait()
    pltpu.async_copy(p_hbm.at[ring.at[1, 0]], buf_b, sem_b)

    def pair(i, carry):
        j0 = 2 * i
        for b, (buf, sem, isem) in enumerate(
                ((buf_a, sem_a, isem_a), (buf_b, sem_b, isem_b))):
            j = j0 + b
            pltpu.make_async_copy(p_hbm.at[pl.ds(0, CHUNK)], buf, sem).wait()
            pltpu.async_copy(idx_hbm.at[wid, j + 2], ring.at[b], isem)
            pltpu.sync_copy(buf, agg_s.at[ring.at[b, 1]], add=True)
            pltpu.make_async_copy(idx_hbm.at[wid, 0], ring.at[b], isem).wait()
            pltpu.async_copy(p_hbm.at[ring.at[b, 0]], buf, sem)
        return carry

    lax.fori_loop(0, NCH // 2, pair, 0)
    pltpu.make_async_copy(p_hbm.at[pl.ds(0, CHUNK)], buf_a, sem_a).wait()
    pltpu.make_async_copy(p_hbm.at[pl.ds(0, CHUNK)], buf_b, sem_b).wait()
    plsc.subcore_barrier()
    pltpu.sync_copy(agg_s.at[rows], out_hbm.at[c, rows])


@functools.cache
def _agg_kernel():
    return pl.kernel(
        _agg_body,
        out_type=jax.ShapeDtypeStruct((NC, NPAD, D), F32),
        mesh=plsc.VectorSubcoreMesh(core_axis_name="c", subcore_axis_name="s",
                                    num_cores=NC, num_subcores=NS),
        scratch_types=[
            pltpu.VMEM((2, 2, CHUNK), jnp.int32),
            pltpu.VMEM((CHUNK, D), F32),
            pltpu.VMEM((CHUNK, D), F32),
            pltpu.SemaphoreType.DMA,
            pltpu.SemaphoreType.DMA,
            pltpu.SemaphoreType.DMA,
            pltpu.SemaphoreType.DMA,
            pltpu.VMEM_SHARED((NPAD, D), F32),
        ],
    )


def _dis_block(deg_ref):
    deg = jnp.sum(deg_ref[...], axis=0) + 1.0
    return lax.rsqrt(deg)


def _row_mask(i):
    r = i * 128 + lax.broadcasted_iota(jnp.int32, (128, 1), 0)
    return r < N


def _pre_body(x_ref, w_ref, deg_ref, o_ref):
    i = pl.program_id(0)
    h = jnp.dot(x_ref[...], w_ref[...], preferred_element_type=F32)
    p = h * _dis_block(deg_ref)
    o_ref[...] = jnp.where(_row_mask(i), p, 0.0)


_pre_kernel = pl.pallas_call(
    _pre_body,
    grid=(NB,),
    in_specs=[
        pl.BlockSpec((128, D), lambda i: (i, 0)),
        pl.BlockSpec((D, D), lambda i: (0, 0)),
        pl.BlockSpec((NW, 128, 1), lambda i: (0, i, 0)),
    ],
    out_specs=pl.BlockSpec((128, D), lambda i: (i, 0)),
    out_shape=jax.ShapeDtypeStruct((NPAD, D), F32),
)


def _mid_body(agg_ref, p1_ref, deg_ref, w_ref, b_ref, o_ref):
    i = pl.program_id(0)
    dis = _dis_block(deg_ref)
    a = agg_ref[0] + agg_ref[1] + p1_ref[...]
    h1 = jnp.maximum(a * dis + b_ref[...], 0.0)
    p2 = jnp.dot(h1, w_ref[...], preferred_element_type=F32) * dis
    o_ref[...] = jnp.where(_row_mask(i), p2, 0.0)


_mid_kernel = pl.pallas_call(
    _mid_body,
    grid=(NB,),
    in_specs=[
        pl.BlockSpec((NC, 128, D), lambda i: (0, i, 0)),
        pl.BlockSpec((128, D), lambda i: (i, 0)),
        pl.BlockSpec((NW, 128, 1), lambda i: (0, i, 0)),
        pl.BlockSpec((D, D), lambda i: (0, 0)),
        pl.BlockSpec((1, D), lambda i: (0, 0)),
    ],
    out_specs=pl.BlockSpec((128, D), lambda i: (i, 0)),
    out_shape=jax.ShapeDtypeStruct((NPAD, D), F32),
)


def _post_body(agg_ref, p2_ref, deg_ref, b_ref, batch_ref, wfc_ref, bfc_ref,
               o_ref, acc, cnt):
    i = pl.program_id(0)

    @pl.when(i == 0)
    def _():
        acc[...] = jnp.zeros((G, D), F32)
        cnt[...] = jnp.zeros((G, D), F32)

    dis = _dis_block(deg_ref)
    a = agg_ref[0] + agg_ref[1] + p2_ref[...]
    h2 = jnp.maximum(a * dis + b_ref[...], 0.0)
    bb = batch_ref[0, 0, :]
    oh = (bb[:, None] == lax.broadcasted_iota(jnp.int32, (128, G), 1))
    oh = oh.astype(F32)
    dn = (((0,), (0,)), ((), ()))
    acc[...] += lax.dot_general(oh, h2, dn, preferred_element_type=F32)
    cnt[...] += lax.dot_general(oh, jnp.ones((128, D), F32), dn,
                                preferred_element_type=F32)

    @pl.when(i == NB - 1)
    def _():
        pooled = acc[...] / jnp.maximum(cnt[...], 1.0)
        z = jnp.sum(pooled * wfc_ref[...], axis=1, keepdims=True)
        z = z + bfc_ref[0, 0]
        o_ref[...] = 1.0 / (1.0 + jnp.exp(-z))


_post_kernel = pl.pallas_call(
    _post_body,
    grid=(NB,),
    in_specs=[
        pl.BlockSpec((NC, 128, D), lambda i: (0, i, 0)),
        pl.BlockSpec((128, D), lambda i: (i, 0)),
        pl.BlockSpec((NW, 128, 1), lambda i: (0, i, 0)),
        pl.BlockSpec((1, D), lambda i: (0, 0)),
        pl.BlockSpec((1, 1, 128), lambda i: (i, 0, 0)),
        pl.BlockSpec((1, D), lambda i: (0, 0)),
        pl.BlockSpec((1, 1), lambda i: (0, 0)),
    ],
    out_specs=pl.BlockSpec((G, 1), lambda i: (0, 0)),
    out_shape=jax.ShapeDtypeStruct((G, 1), F32),
    scratch_shapes=[pltpu.VMEM((G, D), F32), pltpu.VMEM((G, D), F32)],
)


def _pad_edges(edge_index):
    npad = EPAD - E
    pad = N + (jnp.arange(npad, dtype=jnp.int32) % (NPAD - N))
    main = jnp.stack(
        [jnp.concatenate([edge_index[0], pad]).reshape(NW, NCH, CHUNK),
         jnp.concatenate([edge_index[1], pad]).reshape(NW, NCH, CHUNK)],
        axis=2)
    dummy = (N + (jnp.arange(NW * 2 * 2 * CHUNK, dtype=jnp.int32)
                  % (NPAD - N))).reshape(NW, 2, 2, CHUNK)
    return jnp.concatenate([main, dummy], axis=1)


def kernel(x, edge_index, batch, W1, b1, W2, b2, Wfc, bfc):
    idx3 = _pad_edges(edge_index)
    x_pad = jnp.pad(x, ((0, NPAD - N), (0, 0)))
    batch3 = jnp.concatenate(
        [batch, jnp.full((NPAD - N,), G, jnp.int32)]).reshape(NB, 1, 128)
    zeros_n = jnp.zeros((NPAD,), F32)
    zeros_d = jnp.zeros((NPAD, D), F32)
    b1r = b1.reshape(1, D)
    b2r = b2.reshape(1, D)
    wfc_t = Wfc.reshape(1, D)
    bfc_r = bfc.reshape(1, 1)

    deg8 = _deg_kernel()(idx3, zeros_n).reshape(NW, NPAD, 1)
    p1 = _pre_kernel(x_pad, W1, deg8)
    agg1 = _agg_kernel()(p1, idx3, zeros_d)
    p2 = _mid_kernel(agg1, p1, deg8, W2, b1r)
    agg2 = _agg_kernel()(p2, idx3, zeros_d)
    return _post_kernel(agg2, p2, deg8, b2r, batch3, wfc_t, bfc_r)


# speedup vs baseline: 22.6901x; 1.3989x over previous
import functools

import jax
import jax.numpy as jnp
from jax import lax
from jax.experimental import pallas as pl
from jax.experimental.pallas import tpu as pltpu
from jax.experimental.pallas import tpu_sc as plsc

N = 10000
E = 320000
D = 128
G = 64

NC = 2
NS = 16
NW = NC * NS
CHUNK = 128
NCH = 80
NCH_ALL = NCH + 2
EPT = NCH * CHUNK
EPAD = NW * EPT
NPAD = 10240
RPT = NPAD // NS
NB = NPAD // 128
F32 = jnp.float32


def _deg_body(idx_hbm, zeros_hbm, out_hbm, ring, deg_v, isem_a, isem_b):
    c = lax.axis_index("c")
    s = lax.axis_index("s")
    wid = s * NC + c
    pltpu.sync_copy(zeros_hbm, deg_v)
    ones = jnp.full((16,), 1.0, F32)

    pltpu.async_copy(idx_hbm.at[wid, 0], ring.at[0], isem_a)
    pltpu.async_copy(idx_hbm.at[wid, 1], ring.at[1], isem_b)

    def pair(i, carry):
        j0 = 2 * i
        for b, isem in ((0, isem_a), (1, isem_b)):
            j = j0 + b
            pltpu.make_async_copy(idx_hbm.at[wid, 0], ring.at[b], isem).wait()
            for k in range(CHUNK // 16):
                idxv = ring[b, 1, pl.ds(k * 16, 16)]
                plsc.addupdate_scatter(deg_v, [idxv], ones)
            pltpu.async_copy(idx_hbm.at[wid, j + 2], ring.at[b], isem)
        return carry

    lax.fori_loop(0, NCH // 2, pair, 0)
    pltpu.make_async_copy(idx_hbm.at[wid, 0], ring.at[0], isem_a).wait()
    pltpu.make_async_copy(idx_hbm.at[wid, 0], ring.at[1], isem_b).wait()
    pltpu.sync_copy(deg_v, out_hbm.at[wid])


@functools.cache
def _deg_kernel():
    return pl.kernel(
        _deg_body,
        out_type=jax.ShapeDtypeStruct((NW, NPAD), F32),
        mesh=plsc.VectorSubcoreMesh(core_axis_name="c", subcore_axis_name="s",
                                    num_cores=NC, num_subcores=NS),
        scratch_types=[
            pltpu.VMEM((2, 2, CHUNK), jnp.int32),
            pltpu.VMEM((NPAD,), F32),
            pltpu.SemaphoreType.DMA,
            pltpu.SemaphoreType.DMA,
        ],
        compiler_params=pltpu.CompilerParams(needs_layout_passes=False),
    )


def _agg_body(p_hbm, idx_hbm, zeros_hbm, out_hbm,
              ring, buf_a, buf_b, isem_a, isem_b, sem_a, sem_b, agg_s):
    c = lax.axis_index("c")
    s = lax.axis_index("s")
    wid = s * NC + c
    rows = pl.ds(s * RPT, RPT)
    pltpu.sync_copy(zeros_hbm.at[rows], agg_s.at[rows])
    plsc.subcore_barrier()

    pltpu.async_copy(idx_hbm.at[wid, 0], ring.at[0], isem_a)
    pltpu.async_copy(idx_hbm.at[wid, 1], ring.at[1], isem_b)
    pltpu.make_async_copy(idx_hbm.at[wid, 0], ring.at[0], isem_a).wait()
    pltpu.async_copy(p_hbm.at[ring.at[0, 0]], buf_a, sem_a)
    pltpu.make_async_copy(idx_hbm.at[wid, 1], ring.at[1], isem_b).wait()
    pltpu.async_copy(p_hbm.at[ring.at[1, 0]], buf_b, sem_b)

    def pair(i, carry):
        j0 = 2 * i
        for b, (buf, sem, isem) in enumerate(
                ((buf_a, sem_a, isem_a), (buf_b, sem_b, isem_b))):
            j = j0 + b
            pltpu.make_async_copy(p_hbm.at[pl.ds(0, CHUNK)], buf, sem).wait()
            pltpu.async_copy(idx_hbm.at[wid, j + 2], ring.at[b], isem)
            pltpu.sync_copy(buf, agg_s.at[ring.at[b, 1]], add=True)
            pltpu.make_async_copy(idx_hbm.at[wid, 0], ring.at[b], isem).wait()
            pltpu.async_copy(p_hbm.at[ring.at[b, 0]], buf, sem)
        return carry

    lax.fori_loop(0, NCH // 2, pair, 0)
    pltpu.make_async_copy(p_hbm.at[pl.ds(0, CHUNK)], buf_a, sem_a).wait()
    pltpu.make_async_copy(p_hbm.at[pl.ds(0, CHUNK)], buf_b, sem_b).wait()
    plsc.subcore_barrier()
    pltpu.sync_copy(agg_s.at[rows], out_hbm.at[c, rows])


@functools.cache
def _agg_kernel():
    return pl.kernel(
        _agg_body,
        out_type=jax.ShapeDtypeStruct((NC, NPAD, D), F32),
        mesh=plsc.VectorSubcoreMesh(core_axis_name="c", subcore_axis_name="s",
                                    num_cores=NC, num_subcores=NS),
        scratch_types=[
            pltpu.VMEM((2, 2, CHUNK), jnp.int32),
            pltpu.VMEM((CHUNK, D), F32),
            pltpu.VMEM((CHUNK, D), F32),
            pltpu.SemaphoreType.DMA,
            pltpu.SemaphoreType.DMA,
            pltpu.SemaphoreType.DMA,
            pltpu.SemaphoreType.DMA,
            pltpu.VMEM_SHARED((NPAD, D), F32),
        ],
    )


def _dis_block(deg_ref):
    deg = jnp.sum(deg_ref[...], axis=0) + 1.0
    dis = lax.rsqrt(deg).reshape(1, 128)
    return jnp.transpose(dis, (1, 0))


def _row_mask(i):
    r = i * 128 + lax.broadcasted_iota(jnp.int32, (128, 1), 0)
    return r < N


def _pre_body(x_ref, w_ref, deg_ref, o_ref):
    i = pl.program_id(0)
    h = jnp.dot(x_ref[...], w_ref[...], preferred_element_type=F32)
    p = h * _dis_block(deg_ref)
    o_ref[...] = jnp.where(_row_mask(i), p, 0.0)


_pre_kernel = pl.pallas_call(
    _pre_body,
    grid=(NB,),
    in_specs=[
        pl.BlockSpec((128, D), lambda i: (i, 0)),
        pl.BlockSpec((D, D), lambda i: (0, 0)),
        pl.BlockSpec((NW, 128), lambda i: (0, i)),
    ],
    out_specs=pl.BlockSpec((128, D), lambda i: (i, 0)),
    out_shape=jax.ShapeDtypeStruct((NPAD, D), F32),
)


def _mid_body(agg_ref, p1_ref, deg_ref, w_ref, b_ref, o_ref):
    i = pl.program_id(0)
    dis = _dis_block(deg_ref)
    a = agg_ref[0] + agg_ref[1] + p1_ref[...]
    h1 = jnp.maximum(a * dis + b_ref[...], 0.0)
    p2 = jnp.dot(h1, w_ref[...], preferred_element_type=F32) * dis
    o_ref[...] = jnp.where(_row_mask(i), p2, 0.0)


_mid_kernel = pl.pallas_call(
    _mid_body,
    grid=(NB,),
    in_specs=[
        pl.BlockSpec((NC, 128, D), lambda i: (0, i, 0)),
        pl.BlockSpec((128, D), lambda i: (i, 0)),
        pl.BlockSpec((NW, 128), lambda i: (0, i)),
        pl.BlockSpec((D, D), lambda i: (0, 0)),
        pl.BlockSpec((1, D), lambda i: (0, 0)),
    ],
    out_specs=pl.BlockSpec((128, D), lambda i: (i, 0)),
    out_shape=jax.ShapeDtypeStruct((NPAD, D), F32),
)


def _post_body(agg_ref, p2_ref, deg_ref, b_ref, batch_ref, wfc_ref, bfc_ref,
               o_ref, acc, cnt):
    i = pl.program_id(0)

    @pl.when(i == 0)
    def _():
        acc[...] = jnp.zeros((G, D), F32)
        cnt[...] = jnp.zeros((G, D), F32)

    dis = _dis_block(deg_ref)
    a = agg_ref[0] + agg_ref[1] + p2_ref[...]
    h2 = jnp.maximum(a * dis + b_ref[...], 0.0)
    bb = batch_ref[0, 0, :]
    oh = (bb[:, None] == lax.broadcasted_iota(jnp.int32, (128, G), 1))
    oh = oh.astype(F32)
    dn = (((0,), (0,)), ((), ()))
    acc[...] += lax.dot_general(oh, h2, dn, preferred_element_type=F32)
    cnt[...] += lax.dot_general(oh, jnp.ones((128, D), F32), dn,
                                preferred_element_type=F32)

    @pl.when(i == NB - 1)
    def _():
        pooled = acc[...] / jnp.maximum(cnt[...], 1.0)
        z = jnp.sum(pooled * wfc_ref[...], axis=1, keepdims=True)
        z = z + bfc_ref[0, 0]
        o_ref[...] = 1.0 / (1.0 + jnp.exp(-z))


_post_kernel = pl.pallas_call(
    _post_body,
    grid=(NB,),
    in_specs=[
        pl.BlockSpec((NC, 128, D), lambda i: (0, i, 0)),
        pl.BlockSpec((128, D), lambda i: (i, 0)),
        pl.BlockSpec((NW, 128), lambda i: (0, i)),
        pl.BlockSpec((1, D), lambda i: (0, 0)),
        pl.BlockSpec((1, 1, 128), lambda i: (i, 0, 0)),
        pl.BlockSpec((1, D), lambda i: (0, 0)),
        pl.BlockSpec((1, 1), lambda i: (0, 0)),
    ],
    out_specs=pl.BlockSpec((G, 1), lambda i: (0, 0)),
    out_shape=jax.ShapeDtypeStruct((G, 1), F32),
    scratch_shapes=[pltpu.VMEM((G, D), F32), pltpu.VMEM((G, D), F32)],
)


def _pad_edges(edge_index):
    npad = EPAD - E
    pad = N + (jnp.arange(npad, dtype=jnp.int32) % (NPAD - N))
    main = jnp.stack(
        [jnp.concatenate([edge_index[0], pad]).reshape(NW, NCH, CHUNK),
         jnp.concatenate([edge_index[1], pad]).reshape(NW, NCH, CHUNK)],
        axis=2)
    dummy = (N + (jnp.arange(NW * 2 * 2 * CHUNK, dtype=jnp.int32)
                  % (NPAD - N))).reshape(NW, 2, 2, CHUNK)
    return jnp.concatenate([main, dummy], axis=1)


def kernel(x, edge_index, batch, W1, b1, W2, b2, Wfc, bfc):
    idx3 = _pad_edges(edge_index)
    x_pad = jnp.pad(x, ((0, NPAD - N), (0, 0)))
    batch3 = jnp.concatenate(
        [batch, jnp.full((NPAD - N,), G, jnp.int32)]).reshape(NB, 1, 128)
    zeros_n = jnp.zeros((NPAD,), F32)
    zeros_d = jnp.zeros((NPAD, D), F32)
    b1r = b1.reshape(1, D)
    b2r = b2.reshape(1, D)
    wfc_t = Wfc.reshape(1, D)
    bfc_r = bfc.reshape(1, 1)

    deg8 = _deg_kernel()(idx3, zeros_n)
    p1 = _pre_kernel(x_pad, W1, deg8)
    agg1 = _agg_kernel()(p1, idx3, zeros_d)
    p2 = _mid_kernel(agg1, p1, deg8, W2, b1r)
    agg2 = _agg_kernel()(p2, idx3, zeros_d)
    return _post_kernel(agg2, p2, deg8, b2r, batch3, wfc_t, bfc_r)


# speedup vs baseline: 31.1370x; 1.3723x over previous
import functools

import jax
import jax.numpy as jnp
from jax import lax
from jax.experimental import pallas as pl
from jax.experimental.pallas import tpu as pltpu
from jax.experimental.pallas import tpu_sc as plsc

N = 10000
E = 320000
D = 128
G = 64

NC = 2
NS = 16
NW = NC * NS
CHUNK = 128
NCH = 80
NCH_ALL = NCH + 2
EPT = NCH * CHUNK
EPAD = NW * EPT
NPAD = 10240
RPT = NPAD // NS
NB = NPAD // 128
F32 = jnp.float32


def _deg_body(idx_hbm, zeros_hbm, out_hbm, ring, deg_v, isem_a, isem_b):
    c = lax.axis_index("c")
    s = lax.axis_index("s")
    wid = s * NC + c
    pltpu.sync_copy(zeros_hbm, deg_v)
    ones = jnp.full((16,), 1.0, F32)

    pltpu.async_copy(idx_hbm.at[wid, 0], ring.at[0], isem_a)
    pltpu.async_copy(idx_hbm.at[wid, 1], ring.at[1], isem_b)

    def pair(i, carry):
        j0 = 2 * i
        for b, isem in ((0, isem_a), (1, isem_b)):
            j = j0 + b
            pltpu.make_async_copy(idx_hbm.at[wid, 0], ring.at[b], isem).wait()
            for k in range(CHUNK // 16):
                idxv = ring[b, 1, pl.ds(k * 16, 16)]
                plsc.addupdate_scatter(deg_v, [idxv], ones)
            pltpu.async_copy(idx_hbm.at[wid, j + 2], ring.at[b], isem)
        return carry

    lax.fori_loop(0, NCH // 2, pair, 0)
    pltpu.make_async_copy(idx_hbm.at[wid, 0], ring.at[0], isem_a).wait()
    pltpu.make_async_copy(idx_hbm.at[wid, 0], ring.at[1], isem_b).wait()
    pltpu.sync_copy(deg_v, out_hbm.at[wid])


@functools.cache
def _deg_kernel():
    return pl.kernel(
        _deg_body,
        out_type=jax.ShapeDtypeStruct((NW, NPAD), F32),
        mesh=plsc.VectorSubcoreMesh(core_axis_name="c", subcore_axis_name="s",
                                    num_cores=NC, num_subcores=NS),
        scratch_types=[
            pltpu.VMEM((2, 2, CHUNK), jnp.int32),
            pltpu.VMEM((NPAD,), F32),
            pltpu.SemaphoreType.DMA,
            pltpu.SemaphoreType.DMA,
        ],
        compiler_params=pltpu.CompilerParams(needs_layout_passes=False),
    )


def _agg_body(p_hbm, idx_hbm, zeros_hbm, out_hbm,
              ring, buf_a, buf_b, isem_a, isem_b, sem_a, sem_b, agg_s):
    c = lax.axis_index("c")
    s = lax.axis_index("s")
    wid = s * NC + c
    rows = pl.ds(s * RPT, RPT)
    pltpu.sync_copy(zeros_hbm.at[rows], agg_s.at[rows])
    plsc.subcore_barrier()

    pltpu.async_copy(idx_hbm.at[wid, 0], ring.at[0], isem_a)
    pltpu.async_copy(idx_hbm.at[wid, 1], ring.at[1], isem_b)
    pltpu.make_async_copy(idx_hbm.at[wid, 0], ring.at[0], isem_a).wait()
    pltpu.async_copy(p_hbm.at[ring.at[0, 0]], buf_a, sem_a)
    pltpu.make_async_copy(idx_hbm.at[wid, 1], ring.at[1], isem_b).wait()
    pltpu.async_copy(p_hbm.at[ring.at[1, 0]], buf_b, sem_b)

    def pair(i, carry):
        j0 = 2 * i
        for b, (buf, sem, isem) in enumerate(
                ((buf_a, sem_a, isem_a), (buf_b, sem_b, isem_b))):
            j = j0 + b
            pltpu.make_async_copy(p_hbm.at[pl.ds(0, CHUNK)], buf, sem).wait()
            pltpu.async_copy(idx_hbm.at[wid, j + 2], ring.at[b], isem)
            pltpu.sync_copy(buf, agg_s.at[ring.at[b, 1]], add=True)
            pltpu.make_async_copy(idx_hbm.at[wid, 0], ring.at[b], isem).wait()
            pltpu.async_copy(p_hbm.at[ring.at[b, 0]], buf, sem)
        return carry

    lax.fori_loop(0, NCH // 2, pair, 0)
    pltpu.make_async_copy(p_hbm.at[pl.ds(0, CHUNK)], buf_a, sem_a).wait()
    pltpu.make_async_copy(p_hbm.at[pl.ds(0, CHUNK)], buf_b, sem_b).wait()
    plsc.subcore_barrier()
    pltpu.sync_copy(agg_s.at[rows], out_hbm.at[c, rows])


@functools.cache
def _agg_kernel():
    return pl.kernel(
        _agg_body,
        out_type=jax.ShapeDtypeStruct((NC, NPAD, D), F32),
        mesh=plsc.VectorSubcoreMesh(core_axis_name="c", subcore_axis_name="s",
                                    num_cores=NC, num_subcores=NS),
        scratch_types=[
            pltpu.VMEM((2, 2, CHUNK), jnp.int32),
            pltpu.VMEM((CHUNK, D), F32),
            pltpu.VMEM((CHUNK, D), F32),
            pltpu.SemaphoreType.DMA,
            pltpu.SemaphoreType.DMA,
            pltpu.SemaphoreType.DMA,
            pltpu.SemaphoreType.DMA,
            pltpu.VMEM_SHARED((NPAD, D), F32),
        ],
    )


BR = 1024
NBR = NPAD // BR


def _row_mask(i):
    r = i * BR + lax.broadcasted_iota(jnp.int32, (BR, 1), 0)
    return r < N


def _pre_body(x_ref, w_ref, deg_ref, o_ref, dis_ref):
    i = pl.program_id(0)
    h = jnp.dot(x_ref[...], w_ref[...], preferred_element_type=F32)
    deg = jnp.sum(deg_ref[...], axis=0) + 1.0
    dis_row = lax.rsqrt(deg).reshape(1, BR)
    dis = jnp.transpose(dis_row, (1, 0))
    dis_ref[...] = jnp.broadcast_to(dis, (BR, D))
    o_ref[...] = jnp.where(_row_mask(i), h * dis, 0.0)


_pre_kernel = pl.pallas_call(
    _pre_body,
    grid=(NBR,),
    in_specs=[
        pl.BlockSpec((BR, D), lambda i: (i, 0)),
        pl.BlockSpec((D, D), lambda i: (0, 0)),
        pl.BlockSpec((NW, BR), lambda i: (0, i)),
    ],
    out_specs=[
        pl.BlockSpec((BR, D), lambda i: (i, 0)),
        pl.BlockSpec((BR, D), lambda i: (i, 0)),
    ],
    out_shape=[
        jax.ShapeDtypeStruct((NPAD, D), F32),
        jax.ShapeDtypeStruct((NPAD, D), F32),
    ],
)


def _mid_body(agg_ref, p1_ref, dis_ref, w_ref, b_ref, o_ref):
    i = pl.program_id(0)
    dis = dis_ref[...]
    a = agg_ref[0] + agg_ref[1] + p1_ref[...]
    h1 = jnp.maximum(a * dis + b_ref[...], 0.0)
    p2 = jnp.dot(h1, w_ref[...], preferred_element_type=F32) * dis
    o_ref[...] = jnp.where(_row_mask(i), p2, 0.0)


_mid_kernel = pl.pallas_call(
    _mid_body,
    grid=(NBR,),
    in_specs=[
        pl.BlockSpec((NC, BR, D), lambda i: (0, i, 0)),
        pl.BlockSpec((BR, D), lambda i: (i, 0)),
        pl.BlockSpec((BR, D), lambda i: (i, 0)),
        pl.BlockSpec((D, D), lambda i: (0, 0)),
        pl.BlockSpec((1, D), lambda i: (0, 0)),
    ],
    out_specs=pl.BlockSpec((BR, D), lambda i: (i, 0)),
    out_shape=jax.ShapeDtypeStruct((NPAD, D), F32),
)


def _post_body(agg_ref, p2_ref, dis_ref, b_ref, batch_ref, wfc_ref, bfc_ref,
               o_ref, acc, cnt):
    i = pl.program_id(0)

    @pl.when(i == 0)
    def _():
        acc[...] = jnp.zeros((G, D), F32)
        cnt[...] = jnp.zeros((G, 1), F32)

    a = agg_ref[0] + agg_ref[1] + p2_ref[...]
    h2 = jnp.maximum(a * dis_ref[...] + b_ref[...], 0.0)
    bb = batch_ref[0, 0, :]
    oht = (bb[None, :] == lax.broadcasted_iota(jnp.int32, (G, BR), 0))
    oht = oht.astype(F32)
    acc[...] += jnp.dot(oht, h2, preferred_element_type=F32)
    cnt[...] += jnp.sum(oht, axis=1, keepdims=True)

    @pl.when(i == NBR - 1)
    def _():
        pooled = acc[...] / jnp.maximum(cnt[...], 1.0)
        z = jnp.sum(pooled * wfc_ref[...], axis=1, keepdims=True)
        z = z + bfc_ref[0, 0]
        o_ref[...] = 1.0 / (1.0 + jnp.exp(-z))


_post_kernel = pl.pallas_call(
    _post_body,
    grid=(NBR,),
    in_specs=[
        pl.BlockSpec((NC, BR, D), lambda i: (0, i, 0)),
        pl.BlockSpec((BR, D), lambda i: (i, 0)),
        pl.BlockSpec((BR, D), lambda i: (i, 0)),
        pl.BlockSpec((1, D), lambda i: (0, 0)),
        pl.BlockSpec((1, 1, BR), lambda i: (i, 0, 0)),
        pl.BlockSpec((1, D), lambda i: (0, 0)),
        pl.BlockSpec((1, 1), lambda i: (0, 0)),
    ],
    out_specs=pl.BlockSpec((G, 1), lambda i: (0, 0)),
    out_shape=jax.ShapeDtypeStruct((G, 1), F32),
    scratch_shapes=[pltpu.VMEM((G, D), F32), pltpu.VMEM((G, 1), F32)],
)


def _pad_edges(edge_index):
    npad = EPAD - E
    pad = N + (jnp.arange(npad, dtype=jnp.int32) % (NPAD - N))
    main = jnp.stack(
        [jnp.concatenate([edge_index[0], pad]).reshape(NW, NCH, CHUNK),
         jnp.concatenate([edge_index[1], pad]).reshape(NW, NCH, CHUNK)],
        axis=2)
    dummy = (N + (jnp.arange(NW * 2 * 2 * CHUNK, dtype=jnp.int32)
                  % (NPAD - N))).reshape(NW, 2, 2, CHUNK)
    return jnp.concatenate([main, dummy], axis=1)


def kernel(x, edge_index, batch, W1, b1, W2, b2, Wfc, bfc):
    idx3 = _pad_edges(edge_index)
    x_pad = jnp.pad(x, ((0, NPAD - N), (0, 0)))
    batch3 = jnp.concatenate(
        [batch, jnp.full((NPAD - N,), G, jnp.int32)]).reshape(NBR, 1, BR)
    zeros_n = jnp.zeros((NPAD,), F32)
    zeros_d = jnp.zeros((NPAD, D), F32)
    b1r = b1.reshape(1, D)
    b2r = b2.reshape(1, D)
    wfc_t = Wfc.reshape(1, D)
    bfc_r = bfc.reshape(1, 1)

    deg8 = _deg_kernel()(idx3, zeros_n)
    p1, disr = _pre_kernel(x_pad, W1, deg8)
    agg1 = _agg_kernel()(p1, idx3, zeros_d)
    p2 = _mid_kernel(agg1, p1, disr, W2, b1r)
    agg2 = _agg_kernel()(p2, idx3, zeros_d)
    return _post_kernel(agg2, p2, disr, b2r, batch3, wfc_t, bfc_r)
